# baseline (speedup 1.0000x reference)
import functools

import numpy as np
import jax
import jax.numpy as jnp
from jax import lax
from jax.experimental import pallas as pl
from jax.experimental.pallas import tpu as pltpu
from jax.experimental.pallas import tpu_sc as plsc

N_IN = 16
N_H1 = 16
N_H2 = 8
N_OUT = 4
BATCH = 16384
L = 16
NW = 32
ROWS = BATCH // NW
GROUPS = ROWS // L

NP = 164
NP_PAD = 176
W1_, B1_, W2_, B2_, W3_, B3_, R3A_, R3B_ = 0, 64, 80, 112, 120, 152, 156, 160

NE = 185


def _build_param_indices():
    OFF_B, OFF_R, OFF_1 = 128, 156, 184
    ia, ib, sc = [], [], []

    def add(a, b, s):
        ia.append(a); ib.append(b); sc.append(s)

    for h in range(16):
        for k in range(4):
            add(4 * h + k, OFF_1, 2.0)
    for h in range(16):
        add(OFF_B + h, OFF_1, 2.0)
    for j in range(8):
        for k in range(4):
            add(64 + 4 * j + k, OFF_R + (2 * j + k) % 16, 2.0)
    for j in range(8):
        add(OFF_B + 16 + j, OFF_1, 2.0)
    for o in range(4):
        for s in range(8):
            add(96 + 8 * o + s, OFF_R + 16 + s, 2.0)
    for o in range(4):
        add(OFF_B + 24 + o, OFF_1, 2.0)
    for o in range(4):
        add(OFF_R + 24 + o, OFF_1, 1.0)
    for o in range(4):
        add(OFF_R + 24 + o, OFF_1, 2.0)
    assert len(ia) == NP
    while len(ia) < 256:
        add(OFF_1, OFF_1, 0.0)
    return (np.array(ia, np.int32), np.array(ib, np.int32),
            np.array(sc, np.float32))


_IA, _IB, _SC = _build_param_indices()

_MESH = plsc.VectorSubcoreMesh(core_axis_name="c", subcore_axis_name="s")


@functools.partial(
    pl.kernel,
    out_type=jax.ShapeDtypeStruct((BATCH, N_OUT), jnp.float32),
    mesh=_MESH,
    scratch_types=[
        pltpu.VMEM((ROWS // 4, N_IN), jnp.float32),
        pltpu.VMEM((ROWS // 4, N_IN), jnp.float32),
        pltpu.VMEM((ROWS // 2, N_OUT), jnp.float32),
        pltpu.VMEM((ROWS // 2, N_OUT), jnp.float32),
        pltpu.VMEM((NP_PAD * L,), jnp.float32),
        pltpu.VMEM((256,), jnp.float32),
        pltpu.VMEM((256,), jnp.int32),
        pltpu.VMEM((256,), jnp.int32),
        pltpu.VMEM((256,), jnp.float32),
        pltpu.SemaphoreType.DMA,
        pltpu.SemaphoreType.DMA,
        pltpu.SemaphoreType.DMA,
        pltpu.SemaphoreType.DMA,
    ],
    compiler_params=pltpu.CompilerParams(
        needs_layout_passes=False, use_tc_tiling_on_sc=True),
)
def _neat(x_hbm, e_hbm, ia_hbm, ib_hbm, sc_hbm, out_hbm,
          xq0, xq1, ov0, ov1, pv, ev, iav, ibv, scv,
          sx0, sx1, so0, so1):
    wid = lax.axis_index("s") * 2 + lax.axis_index("c")
    base = wid * ROWS
    Q = ROWS // 4
    H2R = ROWS // 2
    GROUPS_Q = Q // L
    xbufs = (xq0, xq1)
    sxs = (sx0, sx1)

    def xcopy(q):
        return pltpu.make_async_copy(
            x_hbm.at[pl.ds(base + q * Q, Q)], xbufs[q % 2], sxs[q % 2])

    def ocopy(h):
        ovh = ov0 if h == 0 else ov1
        soh = so0 if h == 0 else so1
        return pltpu.make_async_copy(
            ovh, out_hbm.at[pl.ds(base + h * H2R, H2R)], soh)

    xcopy(0).start()
    pltpu.sync_copy((e_hbm, ia_hbm, ib_hbm, sc_hbm), (ev, iav, ibv, scv))

    for t in range(NP_PAD // L):
        sl = pl.ds(t * L, L)
        pk = (plsc.load_gather(ev, [iav[sl]])
              * plsc.load_gather(ev, [ibv[sl]]) * scv[sl])
        for l in range(L):
            pv[pl.ds((t * L + l) * L, L)] = jnp.full((L,), pk[l], jnp.float32)

    iota = lax.iota(jnp.int32, L)

    def P(i):
        return pv[pl.ds(i * L, L)]

    def tanh2(a):
        return 1.0 - 2.0 / (jnp.exp(a) + 1.0)

    for q in range(4):
        xcopy(q).wait()
        if q < 3:
            xcopy(q + 1).start()
        xv = xbufs[q % 2]
        ovh = ov0 if q < 2 else ov1

        @plsc.parallel_loop(0, GROUPS_Q, unroll=2)
        def body(g):
            rows = g * L + iota
            cols = [
                plsc.load_gather(xv, [rows, jnp.full((L,), c, jnp.int32)])
                for c in range(N_IN)
            ]
            h1 = []
            for h in range(N_H1):
                a = cols[h % 16] * P(W1_ + 4 * h)
                for k in range(1, 4):
                    a = a + cols[(h + k) % 16] * P(W1_ + 4 * h + k)
                h1.append(tanh2(a + P(B1_ + h)))
            h2 = []
            for j in range(N_H2):
                a = h1[(2 * j) % 16] * P(W2_ + 4 * j)
                for k in range(1, 4):
                    a = a + h1[(2 * j + k) % 16] * P(W2_ + 4 * j + k)
                h2.append(tanh2(a + P(B2_ + j)))
            orows = (q % 2) * Q + rows
            for o in range(N_OUT):
                a = h2[0] * P(W3_ + 8 * o)
                for s in range(1, 8):
                    a = a + h2[s] * P(W3_ + 8 * o + s)
                val = P(R3A_ + o) - P(R3B_ + o) / (jnp.exp(a + P(B3_ + o)) + 1.0)
                plsc.store_scatter(ovh, [orows, jnp.full((L,), o, jnp.int32)], val)

        if q == 1:
            ocopy(0).start()

    ocopy(1).start()
    ocopy(0).wait()
    ocopy(1).wait()


def kernel(x, weights, biases, responses):
    e = jnp.concatenate([weights, biases, responses,
                         jnp.ones((1,), jnp.float32),
                         jnp.zeros((256 - NE,), jnp.float32)])
    return _neat(x, e, jnp.asarray(_IA), jnp.asarray(_IB), jnp.asarray(_SC))


# speedup vs baseline: 1.1663x; 1.1663x over previous
import functools

import numpy as np
import jax
import jax.numpy as jnp
from jax import lax
from jax.experimental import pallas as pl
from jax.experimental.pallas import tpu as pltpu
from jax.experimental.pallas import tpu_sc as plsc

N_IN = 16
N_H1 = 16
N_H2 = 8
N_OUT = 4
BATCH = 16384
L = 16
NW = 32
ROWS = BATCH // NW
GROUPS = ROWS // L

NP = 164
NP_PAD = 176
W1_, B1_, W2_, B2_, W3_, B3_, R3A_, R3B_ = 0, 64, 80, 112, 120, 152, 156, 160

NE = 185


def _build_param_indices():
    OFF_B, OFF_R, OFF_1 = 128, 156, 184
    ia, ib, sc = [], [], []

    def add(a, b, s):
        ia.append(a); ib.append(b); sc.append(s)

    for h in range(16):
        for k in range(4):
            add(4 * h + k, OFF_1, 2.0)
    for h in range(16):
        add(OFF_B + h, OFF_1, 2.0)
    for j in range(8):
        for k in range(4):
            add(64 + 4 * j + k, OFF_R + (2 * j + k) % 16, 2.0)
    for j in range(8):
        add(OFF_B + 16 + j, OFF_1, 2.0)
    for o in range(4):
        for s in range(8):
            add(96 + 8 * o + s, OFF_R + 16 + s, 2.0)
    for o in range(4):
        add(OFF_B + 24 + o, OFF_1, 2.0)
    for o in range(4):
        add(OFF_R + 24 + o, OFF_1, 1.0)
    for o in range(4):
        add(OFF_R + 24 + o, OFF_1, 2.0)
    assert len(ia) == NP
    while len(ia) < 256:
        add(OFF_1, OFF_1, 0.0)
    return (np.array(ia, np.int32), np.array(ib, np.int32),
            np.array(sc, np.float32))


_IA, _IB, _SC = _build_param_indices()

_MESH = plsc.VectorSubcoreMesh(core_axis_name="c", subcore_axis_name="s")


@functools.partial(
    pl.kernel,
    out_type=jax.ShapeDtypeStruct((BATCH, N_OUT), jnp.float32),
    mesh=_MESH,
    scratch_types=[
        pltpu.VMEM((ROWS // 4, N_IN), jnp.float32),
        pltpu.VMEM((ROWS // 4, N_IN), jnp.float32),
        pltpu.VMEM((ROWS // 2, N_OUT), jnp.float32),
        pltpu.VMEM((ROWS // 2, N_OUT), jnp.float32),
        pltpu.VMEM((NP_PAD * L,), jnp.float32),
        pltpu.VMEM((256,), jnp.float32),
        pltpu.VMEM((256,), jnp.int32),
        pltpu.VMEM((256,), jnp.int32),
        pltpu.VMEM((256,), jnp.float32),
        pltpu.SemaphoreType.DMA,
        pltpu.SemaphoreType.DMA,
        pltpu.SemaphoreType.DMA,
        pltpu.SemaphoreType.DMA,
    ],
    compiler_params=pltpu.CompilerParams(
        needs_layout_passes=False, use_tc_tiling_on_sc=True),
)
def _neat(x_hbm, e_hbm, ia_hbm, ib_hbm, sc_hbm, out_hbm,
          xq0, xq1, ov0, ov1, pv, ev, iav, ibv, scv,
          sx0, sx1, so0, so1):
    wid = lax.axis_index("s") * 2 + lax.axis_index("c")
    base = wid * ROWS
    Q = ROWS // 4
    H2R = ROWS // 2
    GROUPS_Q = Q // L
    xbufs = (xq0, xq1)
    sxs = (sx0, sx1)

    def xcopy(q):
        return pltpu.make_async_copy(
            x_hbm.at[pl.ds(base + q * Q, Q)], xbufs[q % 2], sxs[q % 2])

    def ocopy(h):
        ovh = ov0 if h == 0 else ov1
        soh = so0 if h == 0 else so1
        return pltpu.make_async_copy(
            ovh, out_hbm.at[pl.ds(base + h * H2R, H2R)], soh)

    xcopy(0).start()
    pltpu.sync_copy((e_hbm, ia_hbm, ib_hbm, sc_hbm), (ev, iav, ibv, scv))

    for t in range(NP_PAD // L):
        sl = pl.ds(t * L, L)
        pk = (plsc.load_gather(ev, [iav[sl]])
              * plsc.load_gather(ev, [ibv[sl]]) * scv[sl])
        for l in range(L):
            pv[pl.ds((t * L + l) * L, L)] = jnp.full((L,), pk[l], jnp.float32)

    iota = lax.iota(jnp.int32, L)

    def P(i):
        return pv[pl.ds(i * L, L)]

    def tanh2(a):
        return 1.0 - 2.0 / (jnp.exp(a) + 1.0)

    for q in range(4):
        xcopy(q).wait()
        if q < 3:
            xcopy(q + 1).start()
        xv = xbufs[q % 2]
        ovh = ov0 if q < 2 else ov1

        @plsc.parallel_loop(0, GROUPS_Q)
        def body(g):
            rows = g * L + iota
            cols = [
                plsc.load_gather(xv, [rows, jnp.full((L,), c, jnp.int32)])
                for c in range(N_IN)
            ]
            h1 = []
            for h in range(N_H1):
                a = cols[h % 16] * P(W1_ + 4 * h)
                for k in range(1, 4):
                    a = a + cols[(h + k) % 16] * P(W1_ + 4 * h + k)
                h1.append(tanh2(a + P(B1_ + h)))
            h2 = []
            for j in range(N_H2):
                a = h1[(2 * j) % 16] * P(W2_ + 4 * j)
                for k in range(1, 4):
                    a = a + h1[(2 * j + k) % 16] * P(W2_ + 4 * j + k)
                h2.append(tanh2(a + P(B2_ + j)))
            orows = (q % 2) * Q + rows
            for o in range(N_OUT):
                a = h2[0] * P(W3_ + 8 * o)
                for s in range(1, 8):
                    a = a + h2[s] * P(W3_ + 8 * o + s)
                val = P(R3A_ + o) - P(R3B_ + o) / (jnp.exp(a + P(B3_ + o)) + 1.0)
                plsc.store_scatter(ovh, [orows, jnp.full((L,), o, jnp.int32)], val)

        if q == 1:
            ocopy(0).start()

    ocopy(1).start()
    ocopy(0).wait()
    ocopy(1).wait()


def kernel(x, weights, biases, responses):
    e = jnp.concatenate([weights, biases, responses,
                         jnp.ones((1,), jnp.float32),
                         jnp.zeros((256 - NE,), jnp.float32)])
    return _neat(x, e, jnp.asarray(_IA), jnp.asarray(_IB), jnp.asarray(_SC))
